# baseline (speedup 1.0000x reference)
import functools

import jax
import jax.numpy as jnp
from jax import lax
from jax.experimental import pallas as pl
from jax.experimental.pallas import tpu as pltpu
from jax.experimental.pallas import tpu_sc as plsc

N_CH = 16
BATCH = 4096
MAX_LEN = 200
TOTAL = BATCH * MAX_LEN
NW = 32
PER_W = TOTAL // NW
CHUNK = 3200
NCHUNK = PER_W // CHUNK
VPC = CHUNK // 16
TABLE_MAX_KEY = 16


def _sc_body(in_hbm, out_hbm, buf, obuf, sin0, sin1, sout0, sout1):
    core = lax.axis_index("c")
    sub = lax.axis_index("s")
    wid = sub * 2 + core
    base = wid * PER_W
    sins = (sin0, sin1)
    souts = (sout0, sout1)

    def start_in(ci, b):
        off = base + ci * CHUNK
        return [
            pltpu.async_copy(
                in_hbm.at[pl.ds(c * TOTAL + off, CHUNK)],
                buf.at[b, c],
                sins[b],
            )
            for c in range(N_CH)
        ]

    def compute(b):
        def body(i, carry):
            o = i * 16
            acc = buf[b, 0, pl.ds(o, 16)]
            for c in range(1, N_CH):
                acc = acc + buf[b, c, pl.ds(o, 16)] * jnp.int32(c + 1)
            u = lax.bitcast_convert_type(acc, jnp.uint32)
            acc = jnp.where(u <= jnp.uint32(TABLE_MAX_KEY), acc, jnp.int32(0))
            obuf[b, pl.ds(o, 16)] = acc
            return carry

        lax.fori_loop(0, VPC, body, jnp.int32(0), unroll=2)

    pending_in = [None, None]
    pending_out = [None, None]
    pending_in[0] = start_in(0, 0)
    for ci in range(NCHUNK):
        b = ci % 2
        if ci + 1 < NCHUNK:
            pending_in[1 - b] = start_in(ci + 1, 1 - b)
        for cp in pending_in[b]:
            cp.wait()
        if pending_out[b] is not None:
            pending_out[b].wait()
        compute(b)
        pending_out[b] = pltpu.async_copy(
            obuf.at[b],
            out_hbm.at[pl.ds(base + ci * CHUNK, CHUNK)],
            souts[b],
        )
    for b in (0, 1):
        if pending_out[b] is not None:
            pending_out[b].wait()


_mesh = plsc.VectorSubcoreMesh(core_axis_name="c", subcore_axis_name="s")

_sc_call = functools.partial(
    pl.kernel,
    mesh=_mesh,
    out_type=jax.ShapeDtypeStruct((TOTAL,), jnp.int32),
    scratch_types=[
        pltpu.VMEM((2, N_CH, CHUNK), jnp.int32),
        pltpu.VMEM((2, CHUNK), jnp.int32),
        pltpu.SemaphoreType.DMA,
        pltpu.SemaphoreType.DMA,
        pltpu.SemaphoreType.DMA,
        pltpu.SemaphoreType.DMA,
    ],
)(_sc_body)


def kernel(inputs):
    flat = inputs.reshape(N_CH * TOTAL)
    out = _sc_call(flat)
    return out.reshape(BATCH, MAX_LEN)


# speedup vs baseline: 1.9078x; 1.9078x over previous
import functools

import jax
import jax.numpy as jnp
from jax import lax
from jax.experimental import pallas as pl
from jax.experimental.pallas import tpu as pltpu
from jax.experimental.pallas import tpu_sc as plsc

N_CH = 16
BATCH = 4096
MAX_LEN = 200
NW = 32
ROWS_W = BATCH // NW
ROWS_C = 8
NCHUNK = ROWS_W // ROWS_C
NPAIR = NCHUNK // 2
TABLE_MAX_KEY = 16

_WOFFS = tuple(range(0, MAX_LEN - 15, 16)) + (MAX_LEN - 16,)


def _sc_body(in_hbm, out_hbm, buf, obuf, sin0, sin1, sout0, sout1):
    core = lax.axis_index("c")
    sub = lax.axis_index("s")
    wid = sub * 2 + core
    row0 = wid * ROWS_W
    sins = (sin0, sin1)
    souts = (sout0, sout1)

    def in_copies(ci, b):
        r = row0 + ci * ROWS_C
        return [
            pltpu.make_async_copy(
                in_hbm.at[c, pl.ds(r, ROWS_C), :],
                buf.at[b, c],
                sins[b],
            )
            for c in range(N_CH)
        ]

    def out_copy(ci, b):
        r = row0 + ci * ROWS_C
        return pltpu.make_async_copy(
            obuf.at[b],
            out_hbm.at[pl.ds(r, ROWS_C), :],
            souts[b],
        )

    def compute(b):
        def row_body(r, carry):
            for o in _WOFFS:
                acc = buf[b, 0, r, pl.ds(o, 16)]
                for c in range(1, N_CH):
                    acc = acc + buf[b, c, r, pl.ds(o, 16)] * jnp.int32(c + 1)
                u = lax.bitcast_convert_type(acc, jnp.uint32)
                acc = jnp.where(
                    u <= jnp.uint32(TABLE_MAX_KEY), acc, jnp.int32(0)
                )
                obuf[b, r, pl.ds(o, 16)] = acc
            return carry

        lax.fori_loop(0, ROWS_C, row_body, jnp.int32(0))

    for cp in in_copies(0, 0):
        cp.start()
    for cp in in_copies(1, 1):
        cp.start()

    def pair_body(j, carry):
        for b in (0, 1):
            ci = 2 * j + b
            for cp in in_copies(ci, b):
                cp.wait()

            @pl.when(j > 0)
            def _():
                out_copy(ci, b).wait()

            compute(b)
            out_copy(ci, b).start()

            @pl.when(j < NPAIR - 1)
            def _():
                for cp in in_copies(ci + 2, b):
                    cp.start()

        return carry

    lax.fori_loop(0, NPAIR, pair_body, jnp.int32(0))
    out_copy(NCHUNK - 2, 0).wait()
    out_copy(NCHUNK - 1, 1).wait()


_mesh = plsc.VectorSubcoreMesh(core_axis_name="c", subcore_axis_name="s")

_sc_call = functools.partial(
    pl.kernel,
    mesh=_mesh,
    out_type=jax.ShapeDtypeStruct((BATCH, MAX_LEN), jnp.int32),
    scratch_types=[
        pltpu.VMEM((2, N_CH, ROWS_C, MAX_LEN), jnp.int32),
        pltpu.VMEM((2, ROWS_C, MAX_LEN), jnp.int32),
        pltpu.SemaphoreType.DMA,
        pltpu.SemaphoreType.DMA,
        pltpu.SemaphoreType.DMA,
        pltpu.SemaphoreType.DMA,
    ],
)(_sc_body)


def kernel(inputs):
    return _sc_call(inputs)


# speedup vs baseline: 4.1595x; 2.1803x over previous
import functools

import jax
import jax.numpy as jnp
from jax import lax
from jax.experimental import pallas as pl
from jax.experimental.pallas import tpu as pltpu
from jax.experimental.pallas import tpu_sc as plsc

N_CH = 16
BATCH = 4096
MAX_LEN = 200
NW = 32
COLS_W = BATCH // NW
ROWS_C = 8
NCHUNK = MAX_LEN // ROWS_C
NPAIR = (NCHUNK - 1) // 2
TABLE_MAX_KEY = 16


def _sc_body(in_hbm, out_hbm, buf, obuf, sin0, sin1, sout0, sout1):
    core = lax.axis_index("c")
    sub = lax.axis_index("s")
    wid = sub * 2 + core
    col0 = wid * COLS_W
    sins = (sin0, sin1)
    souts = (sout0, sout1)

    def in_copies(ci, b):
        r = ci * ROWS_C
        return [
            pltpu.make_async_copy(
                in_hbm.at[c, pl.ds(r, ROWS_C), pl.ds(col0, COLS_W)],
                buf.at[b, c],
                sins[b],
            )
            for c in range(N_CH)
        ]

    def out_copy(ci, b):
        r = ci * ROWS_C
        return pltpu.make_async_copy(
            obuf.at[b],
            out_hbm.at[pl.ds(r, ROWS_C), pl.ds(col0, COLS_W)],
            souts[b],
        )

    def compute(b):
        def row_body(r, carry):
            for o in range(0, COLS_W, 16):
                acc = buf[b, 0, r, pl.ds(o, 16)]
                for c in range(1, N_CH):
                    acc = acc + buf[b, c, r, pl.ds(o, 16)] * jnp.int32(c + 1)
                u = lax.bitcast_convert_type(acc, jnp.uint32)
                acc = jnp.where(
                    u <= jnp.uint32(TABLE_MAX_KEY), acc, jnp.int32(0)
                )
                obuf[b, r, pl.ds(o, 16)] = acc
            return carry

        lax.fori_loop(0, ROWS_C, row_body, jnp.int32(0))

    for cp in in_copies(0, 0):
        cp.start()
    for cp in in_copies(1, 1):
        cp.start()

    def pair_body(j, carry):
        for b in (0, 1):
            ci = 2 * j + b
            for cp in in_copies(ci, b):
                cp.wait()

            @pl.when(j > 0)
            def _():
                out_copy(ci, b).wait()

            compute(b)
            out_copy(ci, b).start()

            @pl.when(j < NPAIR - (1 if b else 0))
            def _():
                for cp in in_copies(ci + 2, b):
                    cp.start()

        return carry

    lax.fori_loop(0, NPAIR, pair_body, jnp.int32(0))

    last = NCHUNK - 1
    for cp in in_copies(last, 0):
        cp.wait()
    out_copy(last, 0).wait()
    compute(0)
    out_copy(last, 0).start()
    out_copy(last - 1, 1).wait()
    out_copy(last, 0).wait()


_mesh = plsc.VectorSubcoreMesh(core_axis_name="c", subcore_axis_name="s")

_sc_call = functools.partial(
    pl.kernel,
    mesh=_mesh,
    out_type=jax.ShapeDtypeStruct((MAX_LEN, BATCH), jnp.int32),
    scratch_types=[
        pltpu.VMEM((2, N_CH, ROWS_C, COLS_W), jnp.int32),
        pltpu.VMEM((2, ROWS_C, COLS_W), jnp.int32),
        pltpu.SemaphoreType.DMA,
        pltpu.SemaphoreType.DMA,
        pltpu.SemaphoreType.DMA,
        pltpu.SemaphoreType.DMA,
    ],
)(_sc_body)


def kernel(inputs):
    transposed = jnp.transpose(inputs, (0, 2, 1))
    out = _sc_call(transposed)
    return out.T


# speedup vs baseline: 5.0343x; 1.2103x over previous
import functools

import jax
import jax.numpy as jnp
from jax import lax
from jax.experimental import pallas as pl
from jax.experimental.pallas import tpu as pltpu
from jax.experimental.pallas import tpu_sc as plsc

N_CH = 16
BATCH = 4096
MAX_LEN = 200
NW = 32
COLS_W = BATCH // NW
ROWS_C = 8
R_SC = 48
R_TC = MAX_LEN - R_SC
NCHUNK = R_SC // ROWS_C
NPAIR = NCHUNK // 2
TC_ROWS_B = 8
TABLE_MAX_KEY = 16


def _sc_body(in_hbm, out_hbm, buf, obuf, sin0, sin1, sout0, sout1):
    core = lax.axis_index("c")
    sub = lax.axis_index("s")
    wid = sub * 2 + core
    col0 = wid * COLS_W
    sins = (sin0, sin1)
    souts = (sout0, sout1)

    def in_copies(ci, b):
        r = ci * ROWS_C
        return [
            pltpu.make_async_copy(
                in_hbm.at[c, pl.ds(r, ROWS_C), pl.ds(col0, COLS_W)],
                buf.at[b, c],
                sins[b],
            )
            for c in range(N_CH)
        ]

    def out_copy(ci, b):
        r = ci * ROWS_C
        return pltpu.make_async_copy(
            obuf.at[b],
            out_hbm.at[pl.ds(r, ROWS_C), pl.ds(col0, COLS_W)],
            souts[b],
        )

    def compute(b):
        def row_body(r, carry):
            for o in range(0, COLS_W, 16):
                acc = buf[b, 0, r, pl.ds(o, 16)]
                for c in range(1, N_CH):
                    acc = acc + buf[b, c, r, pl.ds(o, 16)] * jnp.int32(c + 1)
                u = lax.bitcast_convert_type(acc, jnp.uint32)
                acc = jnp.where(
                    u <= jnp.uint32(TABLE_MAX_KEY), acc, jnp.int32(0)
                )
                obuf[b, r, pl.ds(o, 16)] = acc
            return carry

        lax.fori_loop(0, ROWS_C, row_body, jnp.int32(0))

    for cp in in_copies(0, 0):
        cp.start()
    for cp in in_copies(1, 1):
        cp.start()

    def pair_body(j, carry):
        for b in (0, 1):
            ci = 2 * j + b
            for cp in in_copies(ci, b):
                cp.wait()

            @pl.when(j > 0)
            def _():
                out_copy(ci, b).wait()

            compute(b)
            out_copy(ci, b).start()

            @pl.when(j < NPAIR - 1)
            def _():
                for cp in in_copies(ci + 2, b):
                    cp.start()

        return carry

    lax.fori_loop(0, NPAIR, pair_body, jnp.int32(0))
    out_copy(NCHUNK - 2, 0).wait()
    out_copy(NCHUNK - 1, 1).wait()


_mesh = plsc.VectorSubcoreMesh(core_axis_name="c", subcore_axis_name="s")

_sc_call = functools.partial(
    pl.kernel,
    mesh=_mesh,
    out_type=jax.ShapeDtypeStruct((R_SC, BATCH), jnp.int32),
    scratch_types=[
        pltpu.VMEM((2, N_CH, ROWS_C, COLS_W), jnp.int32),
        pltpu.VMEM((2, ROWS_C, COLS_W), jnp.int32),
        pltpu.SemaphoreType.DMA,
        pltpu.SemaphoreType.DMA,
        pltpu.SemaphoreType.DMA,
        pltpu.SemaphoreType.DMA,
    ],
)(_sc_body)


def _tc_body(in_ref, out_ref):
    x = in_ref[...]
    acc = x[0]
    for c in range(1, N_CH):
        acc = acc + x[c] * jnp.int32(c + 1)
    u = lax.bitcast_convert_type(acc, jnp.uint32)
    out_ref[...] = jnp.where(
        u <= jnp.uint32(TABLE_MAX_KEY), acc, jnp.int32(0)
    )


_tc_call = pl.pallas_call(
    _tc_body,
    grid=(R_TC // TC_ROWS_B,),
    in_specs=[
        pl.BlockSpec(
            (N_CH, TC_ROWS_B, BATCH),
            lambda i: (0, R_SC // TC_ROWS_B + i, 0),
        )
    ],
    out_specs=pl.BlockSpec((TC_ROWS_B, BATCH), lambda i: (i, 0)),
    out_shape=jax.ShapeDtypeStruct((R_TC, BATCH), jnp.int32),
)


def kernel(inputs):
    transposed = jnp.transpose(inputs, (0, 2, 1))
    top = _sc_call(transposed)
    bottom = _tc_call(transposed)
    return jnp.concatenate([top, bottom], axis=0).T


# speedup vs baseline: 8.2924x; 1.6472x over previous
import functools

import jax
import jax.numpy as jnp
from jax import lax
from jax.experimental import pallas as pl
from jax.experimental.pallas import tpu as pltpu
from jax.experimental.pallas import tpu_sc as plsc

N_CH = 16
BATCH = 4096
MAX_LEN = 200
NW = 32
COLS_W = BATCH // NW
ROWS_C = 8
R_SC = 48
R_TC = MAX_LEN - R_SC
NCHUNK = R_SC // ROWS_C
NPAIR = NCHUNK // 2
TC_ROWS_B = 8
TABLE_MAX_KEY = 16


def _sc_body(in_hbm, out_hbm, buf, obuf, sin0, sin1, sout0, sout1):
    core = lax.axis_index("c")
    sub = lax.axis_index("s")
    wid = sub * 2 + core
    col0 = wid * COLS_W
    sins = (sin0, sin1)
    souts = (sout0, sout1)

    def in_copies(ci, b):
        r = ci * ROWS_C
        return [
            pltpu.make_async_copy(
                in_hbm.at[c, pl.ds(r, ROWS_C), pl.ds(col0, COLS_W)],
                buf.at[b, c],
                sins[b],
            )
            for c in range(N_CH)
        ]

    def out_copy(ci, b):
        r = ci * ROWS_C
        return pltpu.make_async_copy(
            obuf.at[b],
            out_hbm.at[pl.ds(r, ROWS_C), pl.ds(col0, COLS_W)],
            souts[b],
        )

    def compute(b):
        def row_body(r, carry):
            for o in range(0, COLS_W, 16):
                acc = buf[b, 0, r, pl.ds(o, 16)]
                for c in range(1, N_CH):
                    acc = acc + buf[b, c, r, pl.ds(o, 16)] * jnp.int32(c + 1)
                u = lax.bitcast_convert_type(acc, jnp.uint32)
                acc = jnp.where(
                    u <= jnp.uint32(TABLE_MAX_KEY), acc, jnp.int32(0)
                )
                obuf[b, r, pl.ds(o, 16)] = acc
            return carry

        lax.fori_loop(0, ROWS_C, row_body, jnp.int32(0))

    for cp in in_copies(0, 0):
        cp.start()
    for cp in in_copies(1, 1):
        cp.start()

    def pair_body(j, carry):
        for b in (0, 1):
            ci = 2 * j + b
            for cp in in_copies(ci, b):
                cp.wait()

            @pl.when(j > 0)
            def _():
                out_copy(ci, b).wait()

            compute(b)
            out_copy(ci, b).start()

            @pl.when(j < NPAIR - 1)
            def _():
                for cp in in_copies(ci + 2, b):
                    cp.start()

        return carry

    lax.fori_loop(0, NPAIR, pair_body, jnp.int32(0))
    out_copy(NCHUNK - 2, 0).wait()
    out_copy(NCHUNK - 1, 1).wait()


_mesh = plsc.VectorSubcoreMesh(core_axis_name="c", subcore_axis_name="s")

_sc_call = functools.partial(
    pl.kernel,
    mesh=_mesh,
    out_type=jax.ShapeDtypeStruct((R_SC, BATCH), jnp.int32),
    scratch_types=[
        pltpu.VMEM((2, N_CH, ROWS_C, COLS_W), jnp.int32),
        pltpu.VMEM((2, ROWS_C, COLS_W), jnp.int32),
        pltpu.SemaphoreType.DMA,
        pltpu.SemaphoreType.DMA,
        pltpu.SemaphoreType.DMA,
        pltpu.SemaphoreType.DMA,
    ],
)(_sc_body)


def _tc_body(in_ref, out_ref):
    x = in_ref[...]
    acc = x[0]
    for c in range(1, N_CH):
        acc = acc + x[c] * jnp.int32(c + 1)
    u = lax.bitcast_convert_type(acc, jnp.uint32)
    out_ref[...] = jnp.where(
        u <= jnp.uint32(TABLE_MAX_KEY), acc, jnp.int32(0)
    )


_tc_call = pl.pallas_call(
    _tc_body,
    grid=(R_TC // TC_ROWS_B,),
    in_specs=[
        pl.BlockSpec(
            (N_CH, TC_ROWS_B, BATCH),
            lambda i: (0, R_SC // TC_ROWS_B + i, 0),
        )
    ],
    out_specs=pl.BlockSpec((TC_ROWS_B, BATCH), lambda i: (i, 0)),
    out_shape=jax.ShapeDtypeStruct((R_TC, BATCH), jnp.int32),
)


_tc_call_full = pl.pallas_call(
    _tc_body,
    grid=(MAX_LEN // TC_ROWS_B,),
    in_specs=[
        pl.BlockSpec(
            (N_CH, TC_ROWS_B, BATCH),
            lambda i: (0, i, 0),
        )
    ],
    out_specs=pl.BlockSpec((TC_ROWS_B, BATCH), lambda i: (i, 0)),
    out_shape=jax.ShapeDtypeStruct((MAX_LEN, BATCH), jnp.int32),
)


def kernel(inputs):
    transposed = jnp.transpose(inputs, (0, 2, 1))
    return _tc_call_full(transposed).T


# speedup vs baseline: 11.0864x; 1.3369x over previous
import functools

import jax
import jax.numpy as jnp
from jax import lax
from jax.experimental import pallas as pl
from jax.experimental.pallas import tpu as pltpu
from jax.experimental.pallas import tpu_sc as plsc

N_CH = 16
BATCH = 4096
MAX_LEN = 200
NW = 32
COLS_W = BATCH // NW
ROWS_C = 8
R_SC = 48
R_TC = MAX_LEN - R_SC
NCHUNK = R_SC // ROWS_C
NPAIR = NCHUNK // 2
TC_ROWS_B = 8
TABLE_MAX_KEY = 16


def _sc_body(in_hbm, out_hbm, buf, obuf, sin0, sin1, sout0, sout1):
    core = lax.axis_index("c")
    sub = lax.axis_index("s")
    wid = sub * 2 + core
    col0 = wid * COLS_W
    sins = (sin0, sin1)
    souts = (sout0, sout1)

    def in_copies(ci, b):
        r = ci * ROWS_C
        return [
            pltpu.make_async_copy(
                in_hbm.at[c, pl.ds(r, ROWS_C), pl.ds(col0, COLS_W)],
                buf.at[b, c],
                sins[b],
            )
            for c in range(N_CH)
        ]

    def out_copy(ci, b):
        r = ci * ROWS_C
        return pltpu.make_async_copy(
            obuf.at[b],
            out_hbm.at[pl.ds(r, ROWS_C), pl.ds(col0, COLS_W)],
            souts[b],
        )

    def compute(b):
        def row_body(r, carry):
            for o in range(0, COLS_W, 16):
                acc = buf[b, 0, r, pl.ds(o, 16)]
                for c in range(1, N_CH):
                    acc = acc + buf[b, c, r, pl.ds(o, 16)] * jnp.int32(c + 1)
                u = lax.bitcast_convert_type(acc, jnp.uint32)
                acc = jnp.where(
                    u <= jnp.uint32(TABLE_MAX_KEY), acc, jnp.int32(0)
                )
                obuf[b, r, pl.ds(o, 16)] = acc
            return carry

        lax.fori_loop(0, ROWS_C, row_body, jnp.int32(0))

    for cp in in_copies(0, 0):
        cp.start()
    for cp in in_copies(1, 1):
        cp.start()

    def pair_body(j, carry):
        for b in (0, 1):
            ci = 2 * j + b
            for cp in in_copies(ci, b):
                cp.wait()

            @pl.when(j > 0)
            def _():
                out_copy(ci, b).wait()

            compute(b)
            out_copy(ci, b).start()

            @pl.when(j < NPAIR - 1)
            def _():
                for cp in in_copies(ci + 2, b):
                    cp.start()

        return carry

    lax.fori_loop(0, NPAIR, pair_body, jnp.int32(0))
    out_copy(NCHUNK - 2, 0).wait()
    out_copy(NCHUNK - 1, 1).wait()


_mesh = plsc.VectorSubcoreMesh(core_axis_name="c", subcore_axis_name="s")

_sc_call = functools.partial(
    pl.kernel,
    mesh=_mesh,
    out_type=jax.ShapeDtypeStruct((R_SC, BATCH), jnp.int32),
    scratch_types=[
        pltpu.VMEM((2, N_CH, ROWS_C, COLS_W), jnp.int32),
        pltpu.VMEM((2, ROWS_C, COLS_W), jnp.int32),
        pltpu.SemaphoreType.DMA,
        pltpu.SemaphoreType.DMA,
        pltpu.SemaphoreType.DMA,
        pltpu.SemaphoreType.DMA,
    ],
)(_sc_body)


def _tc_body(in_ref, out_ref):
    x = in_ref[...]
    acc = x[0]
    for c in range(1, N_CH):
        acc = acc + x[c] * jnp.int32(c + 1)
    u = lax.bitcast_convert_type(acc, jnp.uint32)
    out_ref[...] = jnp.where(
        u <= jnp.uint32(TABLE_MAX_KEY), acc, jnp.int32(0)
    )


_tc_call = pl.pallas_call(
    _tc_body,
    grid=(R_TC // TC_ROWS_B,),
    in_specs=[
        pl.BlockSpec(
            (N_CH, TC_ROWS_B, BATCH),
            lambda i: (0, R_SC // TC_ROWS_B + i, 0),
        )
    ],
    out_specs=pl.BlockSpec((TC_ROWS_B, BATCH), lambda i: (i, 0)),
    out_shape=jax.ShapeDtypeStruct((R_TC, BATCH), jnp.int32),
)


TC_ROWS_FULL = 40

_tc_call_full = pl.pallas_call(
    _tc_body,
    grid=(MAX_LEN // TC_ROWS_FULL,),
    in_specs=[
        pl.BlockSpec(
            (N_CH, TC_ROWS_FULL, BATCH),
            lambda i: (0, i, 0),
        )
    ],
    out_specs=pl.BlockSpec((TC_ROWS_FULL, BATCH), lambda i: (i, 0)),
    out_shape=jax.ShapeDtypeStruct((MAX_LEN, BATCH), jnp.int32),
)


def kernel(inputs):
    transposed = jnp.transpose(inputs, (0, 2, 1))
    return _tc_call_full(transposed).T
